# Initial kernel scaffold; baseline (speedup 1.0000x reference)
import jax
import jax.numpy as jnp
from jax.experimental import pallas as pl


def kernel(inputs, edge_index, weight, bias):
    raise NotImplementedError("write your pallas kernel here")


# speedup vs baseline: 7.7711x; 7.7711x over previous
import functools

import jax
import jax.numpy as jnp
from jax import lax
from jax.experimental import pallas as pl
from jax.experimental.pallas import tpu as pltpu
from jax.experimental.pallas import tpu_sc as plsc

N_NODES = 10000
N_EDGES = 320000
F = 128

NC = 2
NS = 16
NW = NC * NS

R_MAIN = 632
R_LAST = N_NODES - (NS - 1) * R_MAIN
EDGES_PER_TILE = N_EDGES // NW
K = 80
CHUNKS = EDGES_PER_TILE // K


def _sc_aggregate_body(x_hbm, src_hbm, dst_hbm, out_hbm,
                       acc_sh, src_v, dst_v, rows_v, sem):
    c = lax.axis_index("c")
    s = lax.axis_index("s")
    wid = c * NS + s

    zero16 = jnp.zeros((16,), jnp.float32)

    def _zrow(i, carry):
        def _zcol(k, carry2):
            rows_v[i, pl.ds(k * 16, 16)] = zero16
            return carry2
        return lax.fori_loop(0, F // 16, _zcol, carry)

    lax.fori_loop(0, K, _zrow, 0)

    base = s * R_MAIN

    def _zero_acc(nrows):
        nfull = nrows // K
        rem = nrows - nfull * K

        def _zacc(i, carry):
            pltpu.sync_copy(rows_v, acc_sh.at[pl.ds(base + i * K, K)])
            return carry

        lax.fori_loop(0, nfull, _zacc, 0)
        if rem:
            pltpu.sync_copy(rows_v.at[pl.ds(0, rem)],
                            acc_sh.at[pl.ds(base + nfull * K, rem)])

    @pl.when(s < NS - 1)
    def _():
        _zero_acc(R_MAIN)

    @pl.when(s == NS - 1)
    def _():
        _zero_acc(R_LAST)

    plsc.subcore_barrier()

    pltpu.sync_copy(src_hbm.at[wid], src_v)
    pltpu.sync_copy(dst_hbm.at[wid], dst_v)

    def _step(j, carry):
        pltpu.async_copy(x_hbm.at[src_v.at[j]], rows_v, sem).wait()
        pltpu.sync_copy(rows_v, acc_sh.at[dst_v.at[j]], add=True)
        return carry

    lax.fori_loop(0, CHUNKS, _step, 0)
    plsc.subcore_barrier()

    @pl.when(s < NS - 1)
    def _():
        pltpu.sync_copy(acc_sh.at[pl.ds(base, R_MAIN)],
                        out_hbm.at[c, pl.ds(base, R_MAIN)])

    @pl.when(s == NS - 1)
    def _():
        pltpu.sync_copy(acc_sh.at[pl.ds(base, R_LAST)],
                        out_hbm.at[c, pl.ds(base, R_LAST)])


@jax.jit
def _sc_aggregate(x, src, dst):
    mesh = plsc.VectorSubcoreMesh(core_axis_name="c", subcore_axis_name="s")
    return pl.kernel(
        _sc_aggregate_body,
        out_type=jax.ShapeDtypeStruct((NC, N_NODES, F), jnp.float32),
        mesh=mesh,
        scratch_types=[
            pltpu.VMEM_SHARED((N_NODES, F), jnp.float32),
            pltpu.VMEM((CHUNKS, K), jnp.int32),
            pltpu.VMEM((CHUNKS, K), jnp.int32),
            pltpu.VMEM((K, F), jnp.float32),
            pltpu.SemaphoreType.DMA,
        ],
    )(x, src, dst)


def _tc_finish_body(acc_ref, w_ref, b_ref, o_ref):
    a = acc_ref[0] + acc_ref[1]
    y = jnp.dot(a, w_ref[...], preferred_element_type=jnp.float32)
    o_ref[...] = jnp.tanh(y + b_ref[...])


@jax.jit
def _tc_finish(agg, weight, bias):
    blk = 1000
    return pl.pallas_call(
        _tc_finish_body,
        grid=(N_NODES // blk,),
        in_specs=[
            pl.BlockSpec((NC, blk, F), lambda i: (0, i, 0)),
            pl.BlockSpec((F, F), lambda i: (0, 0)),
            pl.BlockSpec((1, F), lambda i: (0, 0)),
        ],
        out_specs=pl.BlockSpec((blk, F), lambda i: (i, 0)),
        out_shape=jax.ShapeDtypeStruct((N_NODES, F), jnp.float32),
    )(agg, weight, bias.reshape(1, F))


def kernel(inputs, edge_index, weight, bias):
    src = edge_index[0].astype(jnp.int32).reshape(NW, CHUNKS, K)
    dst = edge_index[1].astype(jnp.int32).reshape(NW, CHUNKS, K)
    agg = _sc_aggregate(inputs, src, dst)
    return _tc_finish(agg, weight, bias)


# speedup vs baseline: 11.4878x; 1.4783x over previous
import functools

import jax
import jax.numpy as jnp
from jax import lax
from jax.experimental import pallas as pl
from jax.experimental.pallas import tpu as pltpu
from jax.experimental.pallas import tpu_sc as plsc

N_NODES = 10000
N_EDGES = 320000
F = 128

NC = 2
NS = 16
NW = NC * NS

R_MAIN = 632
R_LAST = N_NODES - (NS - 1) * R_MAIN
EDGES_PER_TILE = N_EDGES // NW
K = 80
CHUNKS = EDGES_PER_TILE // K
G = 25
NG = CHUNKS // G


def _sc_aggregate_body(x_hbm, src_hbm, dst_hbm, out_hbm,
                       acc_sh, src_v, dst_v, rows_v, rows2_v, sem, sem2):
    c = lax.axis_index("c")
    s = lax.axis_index("s")
    wid = c * NS + s

    zero16 = jnp.zeros((16,), jnp.float32)

    def _zrow(i, carry):
        def _zcol(k, carry2):
            rows_v[i, pl.ds(k * 16, 16)] = zero16
            return carry2
        return lax.fori_loop(0, F // 16, _zcol, carry)

    lax.fori_loop(0, K, _zrow, 0)

    base = s * R_MAIN

    def _zero_acc(nrows):
        nfull = nrows // K
        rem = nrows - nfull * K

        def _zacc(i, carry):
            pltpu.sync_copy(rows_v, acc_sh.at[pl.ds(base + i * K, K)])
            return carry

        lax.fori_loop(0, nfull, _zacc, 0)
        if rem:
            pltpu.sync_copy(rows_v.at[pl.ds(0, rem)],
                            acc_sh.at[pl.ds(base + nfull * K, rem)])

    @pl.when(s < NS - 1)
    def _():
        _zero_acc(R_MAIN)

    @pl.when(s == NS - 1)
    def _():
        _zero_acc(R_LAST)

    plsc.subcore_barrier()

    def _start(j, buf, s):
        pltpu.async_copy(x_hbm.at[src_v.at[j]], buf, s)

    def _finish(j, buf, s):
        pltpu.make_async_copy(x_hbm.at[src_v.at[j]], buf, s).wait()
        pltpu.sync_copy(buf, acc_sh.at[dst_v.at[j]], add=True)

    @pl.loop(0, NG)
    def _(g):
        pltpu.sync_copy(src_hbm.at[wid, g], src_v)
        pltpu.sync_copy(dst_hbm.at[wid, g], dst_v)
        _start(0, rows_v, sem)

        @pl.loop(0, G - 1, step=2)
        def _(jb):
            _start(jb + 1, rows2_v, sem2)
            _finish(jb, rows_v, sem)
            _start(jb + 2, rows_v, sem)
            _finish(jb + 1, rows2_v, sem2)

        _finish(G - 1, rows_v, sem)

    plsc.subcore_barrier()

    @pl.when(s < NS - 1)
    def _():
        pltpu.sync_copy(acc_sh.at[pl.ds(base, R_MAIN)],
                        out_hbm.at[c, pl.ds(base, R_MAIN)])

    @pl.when(s == NS - 1)
    def _():
        pltpu.sync_copy(acc_sh.at[pl.ds(base, R_LAST)],
                        out_hbm.at[c, pl.ds(base, R_LAST)])


@jax.jit
def _sc_aggregate(x, src, dst):
    mesh = plsc.VectorSubcoreMesh(core_axis_name="c", subcore_axis_name="s")
    return pl.kernel(
        _sc_aggregate_body,
        out_type=jax.ShapeDtypeStruct((NC, N_NODES, F), jnp.float32),
        mesh=mesh,
        scratch_types=[
            pltpu.VMEM_SHARED((N_NODES, F), jnp.float32),
            pltpu.VMEM((G, K), jnp.int32),
            pltpu.VMEM((G, K), jnp.int32),
            pltpu.VMEM((K, F), jnp.float32),
            pltpu.VMEM((K, F), jnp.float32),
            pltpu.SemaphoreType.DMA,
            pltpu.SemaphoreType.DMA,
        ],
    )(x, src, dst)


def _tc_finish_body(acc_ref, w_ref, b_ref, o_ref):
    a = acc_ref[0] + acc_ref[1]
    y = jnp.dot(a, w_ref[...], preferred_element_type=jnp.float32)
    o_ref[...] = jnp.tanh(y + b_ref[...])


@jax.jit
def _tc_finish(agg, weight, bias):
    blk = 1000
    return pl.pallas_call(
        _tc_finish_body,
        grid=(N_NODES // blk,),
        in_specs=[
            pl.BlockSpec((NC, blk, F), lambda i: (0, i, 0)),
            pl.BlockSpec((F, F), lambda i: (0, 0)),
            pl.BlockSpec((1, F), lambda i: (0, 0)),
        ],
        out_specs=pl.BlockSpec((blk, F), lambda i: (i, 0)),
        out_shape=jax.ShapeDtypeStruct((N_NODES, F), jnp.float32),
    )(agg, weight, bias.reshape(1, F))


def kernel(inputs, edge_index, weight, bias):
    src = edge_index[0].astype(jnp.int32).reshape(NW, NG, G, K)
    dst = edge_index[1].astype(jnp.int32).reshape(NW, NG, G, K)
    agg = _sc_aggregate(inputs, src, dst)
    return _tc_finish(agg, weight, bias)


# speedup vs baseline: 12.4377x; 1.0827x over previous
import functools

import jax
import jax.numpy as jnp
from jax import lax
from jax.experimental import pallas as pl
from jax.experimental.pallas import tpu as pltpu
from jax.experimental.pallas import tpu_sc as plsc

N_NODES = 10000
N_EDGES = 320000
F = 128

NC = 2
NS = 16
NW = NC * NS

R_MAIN = 632
R_LAST = N_NODES - (NS - 1) * R_MAIN
EDGES_PER_TILE = N_EDGES // NW
K = 125
CHUNKS = EDGES_PER_TILE // K
G = 20
NG = CHUNKS // G
Z = 120


def _sc_aggregate_body(x_hbm, src_hbm, dst_hbm, out_hbm,
                       acc_sh, src_v, dst_v, rows_v, rows2_v, sem, sem2):
    c = lax.axis_index("c")
    s = lax.axis_index("s")
    wid = c * NS + s

    zero16 = jnp.zeros((16,), jnp.float32)

    def _zrow(i, carry):
        def _zcol(k, carry2):
            rows_v[i, pl.ds(k * 16, 16)] = zero16
            return carry2
        return lax.fori_loop(0, F // 16, _zcol, carry)

    lax.fori_loop(0, Z, _zrow, 0)

    base = s * R_MAIN

    def _zero_acc(nrows):
        nfull = nrows // Z
        rem = nrows - nfull * Z

        def _zacc(i, carry):
            pltpu.sync_copy(rows_v.at[pl.ds(0, Z)],
                            acc_sh.at[pl.ds(base + i * Z, Z)])
            return carry

        lax.fori_loop(0, nfull, _zacc, 0)
        if rem:
            pltpu.sync_copy(rows_v.at[pl.ds(0, rem)],
                            acc_sh.at[pl.ds(base + nfull * Z, rem)])

    @pl.when(s < NS - 1)
    def _():
        _zero_acc(R_MAIN)

    @pl.when(s == NS - 1)
    def _():
        _zero_acc(R_LAST)

    plsc.subcore_barrier()

    def _start(j, buf, s):
        pltpu.async_copy(x_hbm.at[src_v.at[j]], buf, s)

    def _finish(j, buf, s):
        pltpu.make_async_copy(x_hbm.at[src_v.at[j]], buf, s).wait()
        pltpu.sync_copy(buf, acc_sh.at[dst_v.at[j]], add=True)

    @pl.loop(0, NG)
    def _(g):
        pltpu.sync_copy(src_hbm.at[wid, g], src_v)
        pltpu.sync_copy(dst_hbm.at[wid, g], dst_v)
        _start(0, rows_v, sem)

        @pl.loop(0, G - 2, step=2)
        def _(jb):
            _start(jb + 1, rows2_v, sem2)
            _finish(jb, rows_v, sem)
            _start(jb + 2, rows_v, sem)
            _finish(jb + 1, rows2_v, sem2)

        _start(G - 1, rows2_v, sem2)
        _finish(G - 2, rows_v, sem)
        _finish(G - 1, rows2_v, sem2)

    plsc.subcore_barrier()

    @pl.when(s < NS - 1)
    def _():
        pltpu.sync_copy(acc_sh.at[pl.ds(base, R_MAIN)],
                        out_hbm.at[c, pl.ds(base, R_MAIN)])

    @pl.when(s == NS - 1)
    def _():
        pltpu.sync_copy(acc_sh.at[pl.ds(base, R_LAST)],
                        out_hbm.at[c, pl.ds(base, R_LAST)])


@jax.jit
def _sc_aggregate(x, src, dst):
    mesh = plsc.VectorSubcoreMesh(core_axis_name="c", subcore_axis_name="s")
    return pl.kernel(
        _sc_aggregate_body,
        out_type=jax.ShapeDtypeStruct((NC, N_NODES, F), jnp.float32),
        mesh=mesh,
        scratch_types=[
            pltpu.VMEM_SHARED((N_NODES, F), jnp.float32),
            pltpu.VMEM((G, K), jnp.int32),
            pltpu.VMEM((G, K), jnp.int32),
            pltpu.VMEM((K, F), jnp.float32),
            pltpu.VMEM((K, F), jnp.float32),
            pltpu.SemaphoreType.DMA,
            pltpu.SemaphoreType.DMA,
        ],
    )(x, src, dst)


def _tc_finish_body(acc_ref, w_ref, b_ref, o_ref):
    a = acc_ref[0] + acc_ref[1]
    y = jnp.dot(a, w_ref[...], preferred_element_type=jnp.float32)
    o_ref[...] = jnp.tanh(y + b_ref[...])


@jax.jit
def _tc_finish(agg, weight, bias):
    blk = 1000
    return pl.pallas_call(
        _tc_finish_body,
        grid=(N_NODES // blk,),
        in_specs=[
            pl.BlockSpec((NC, blk, F), lambda i: (0, i, 0)),
            pl.BlockSpec((F, F), lambda i: (0, 0)),
            pl.BlockSpec((1, F), lambda i: (0, 0)),
        ],
        out_specs=pl.BlockSpec((blk, F), lambda i: (i, 0)),
        out_shape=jax.ShapeDtypeStruct((N_NODES, F), jnp.float32),
    )(agg, weight, bias.reshape(1, F))


def kernel(inputs, edge_index, weight, bias):
    src = edge_index[0].astype(jnp.int32).reshape(NW, NG, G, K)
    dst = edge_index[1].astype(jnp.int32).reshape(NW, NG, G, K)
    agg = _sc_aggregate(inputs, src, dst)
    return _tc_finish(agg, weight, bias)


# speedup vs baseline: 12.6861x; 1.0200x over previous
import functools

import jax
import jax.numpy as jnp
from jax import lax
from jax.experimental import pallas as pl
from jax.experimental.pallas import tpu as pltpu
from jax.experimental.pallas import tpu_sc as plsc

N_NODES = 10000
N_EDGES = 320000
F = 128

NC = 2
NS = 16
NW = NC * NS

R_MAIN = 632
R_LAST = N_NODES - (NS - 1) * R_MAIN
K = 128
CH_MAIN = 78
N_EXTRA = N_EDGES // K - NW * CH_MAIN
G = 26
NG = CH_MAIN // G
GSZ = G * K
Z = 120


def _sc_aggregate_body(x_hbm, src_hbm, dst_hbm, out_hbm,
                       acc_sh, src_v, dst_v, dstk_v, rows_v, rows2_v,
                       sem, sem2):
    c = lax.axis_index("c")
    s = lax.axis_index("s")
    wid = c * NS + s

    zero16 = jnp.zeros((16,), jnp.float32)

    def _zrow(i, carry):
        def _zcol(k, carry2):
            rows_v[i, pl.ds(k * 16, 16)] = zero16
            return carry2
        return lax.fori_loop(0, F // 16, _zcol, carry)

    lax.fori_loop(0, Z, _zrow, 0)

    base = s * R_MAIN

    def _zero_acc(nrows):
        nfull = nrows // Z
        rem = nrows - nfull * Z

        def _zacc(i, carry):
            pltpu.sync_copy(rows_v.at[pl.ds(0, Z)],
                            acc_sh.at[pl.ds(base + i * Z, Z)])
            return carry

        lax.fori_loop(0, nfull, _zacc, 0)
        if rem:
            pltpu.sync_copy(rows_v.at[pl.ds(0, rem)],
                            acc_sh.at[pl.ds(base + nfull * Z, rem)])

    @pl.when(s < NS - 1)
    def _():
        _zero_acc(R_MAIN)

    @pl.when(s == NS - 1)
    def _():
        _zero_acc(R_LAST)

    plsc.subcore_barrier()

    ebase = (wid * CH_MAIN + jnp.minimum(wid, N_EXTRA)) * K

    def _start(j, buf, s):
        pltpu.async_copy(x_hbm.at[src_v.at[pl.ds(j * K, K)]], buf, s)

    def _finish(j, buf, s):
        pltpu.make_async_copy(x_hbm.at[src_v.at[pl.ds(j * K, K)]], buf, s
                              ).wait()
        for i in range(K // 16):
            dstk_v[pl.ds(i * 16, 16)] = dst_v[pl.ds(j * K + i * 16, 16)]
        pltpu.sync_copy(buf, acc_sh.at[dstk_v], add=True)

    @pl.loop(0, NG)
    def _(g):
        pltpu.sync_copy(src_hbm.at[pl.ds(ebase + g * GSZ, GSZ)], src_v)
        pltpu.sync_copy(dst_hbm.at[pl.ds(ebase + g * GSZ, GSZ)], dst_v)
        _start(0, rows_v, sem)

        @pl.loop(0, G - 2, step=2)
        def _(jb):
            _start(jb + 1, rows2_v, sem2)
            _finish(jb, rows_v, sem)
            _start(jb + 2, rows_v, sem)
            _finish(jb + 1, rows2_v, sem2)

        _start(G - 1, rows2_v, sem2)
        _finish(G - 2, rows_v, sem)
        _finish(G - 1, rows2_v, sem2)

    @pl.when(wid < N_EXTRA)
    def _():
        pltpu.sync_copy(src_hbm.at[pl.ds(ebase + CH_MAIN * K, K)],
                        src_v.at[pl.ds(0, K)])
        pltpu.sync_copy(dst_hbm.at[pl.ds(ebase + CH_MAIN * K, K)],
                        dst_v.at[pl.ds(0, K)])
        _start(0, rows_v, sem)
        _finish(0, rows_v, sem)

    plsc.subcore_barrier()

    @pl.when(s < NS - 1)
    def _():
        pltpu.sync_copy(acc_sh.at[pl.ds(base, R_MAIN)],
                        out_hbm.at[c, pl.ds(base, R_MAIN)])

    @pl.when(s == NS - 1)
    def _():
        pltpu.sync_copy(acc_sh.at[pl.ds(base, R_LAST)],
                        out_hbm.at[c, pl.ds(base, R_LAST)])


@jax.jit
def _sc_aggregate(x, src, dst):
    mesh = plsc.VectorSubcoreMesh(core_axis_name="c", subcore_axis_name="s")
    return pl.kernel(
        _sc_aggregate_body,
        out_type=jax.ShapeDtypeStruct((NC, N_NODES, F), jnp.float32),
        mesh=mesh,
        scratch_types=[
            pltpu.VMEM_SHARED((N_NODES, F), jnp.float32),
            pltpu.VMEM((GSZ,), jnp.int32),
            pltpu.VMEM((GSZ,), jnp.int32),
            pltpu.VMEM((K,), jnp.int32),
            pltpu.VMEM((K, F), jnp.float32),
            pltpu.VMEM((K, F), jnp.float32),
            pltpu.SemaphoreType.DMA,
            pltpu.SemaphoreType.DMA,
        ],
    )(x, src, dst)


def _tc_finish_body(acc_ref, w_ref, b_ref, o_ref):
    a = acc_ref[0] + acc_ref[1]
    y = jnp.dot(a, w_ref[...], preferred_element_type=jnp.float32)
    o_ref[...] = jnp.tanh(y + b_ref[...])


@jax.jit
def _tc_finish(agg, weight, bias):
    blk = 1000
    return pl.pallas_call(
        _tc_finish_body,
        grid=(N_NODES // blk,),
        in_specs=[
            pl.BlockSpec((NC, blk, F), lambda i: (0, i, 0)),
            pl.BlockSpec((F, F), lambda i: (0, 0)),
            pl.BlockSpec((1, F), lambda i: (0, 0)),
        ],
        out_specs=pl.BlockSpec((blk, F), lambda i: (i, 0)),
        out_shape=jax.ShapeDtypeStruct((N_NODES, F), jnp.float32),
    )(agg, weight, bias.reshape(1, F))


def kernel(inputs, edge_index, weight, bias):
    src = edge_index[0].astype(jnp.int32)
    dst = edge_index[1].astype(jnp.int32)
    agg = _sc_aggregate(inputs, src, dst)
    return _tc_finish(agg, weight, bias)


# speedup vs baseline: 13.1220x; 1.0344x over previous
import functools

import jax
import jax.numpy as jnp
from jax import lax
from jax.experimental import pallas as pl
from jax.experimental.pallas import tpu as pltpu
from jax.experimental.pallas import tpu_sc as plsc

N_NODES = 10000
N_EDGES = 320000
F = 128

NC = 2
NS = 16
NW = NC * NS

R_MAIN = 632
R_LAST = N_NODES - (NS - 1) * R_MAIN
K = 128
N_CHUNKS = N_EDGES // K
CH_TILE = 80
SC0_CHUNKS = 1248
G = 16
Z = 120


def _sc_aggregate_body(x_hbm, e_hbm, out_hbm,
                       acc_sh, src_v, dst_v, rows_v, rows2_v,
                       sem, sem2):
    c = lax.axis_index("c")
    s = lax.axis_index("s")
    wid = c * NS + s

    zero16 = jnp.zeros((16,), jnp.float32)

    def _zrow(i, carry):
        def _zcol(k, carry2):
            rows_v[i, pl.ds(k * 16, 16)] = zero16
            return carry2
        return lax.fori_loop(0, F // 16, _zcol, carry)

    lax.fori_loop(0, Z, _zrow, 0)

    base = s * R_MAIN

    def _zero_acc(nrows):
        nfull = nrows // Z
        rem = nrows - nfull * Z

        def _zacc(i, carry):
            pltpu.sync_copy(rows_v.at[pl.ds(0, Z)],
                            acc_sh.at[pl.ds(base + i * Z, Z)])
            return carry

        lax.fori_loop(0, nfull, _zacc, 0)
        if rem:
            pltpu.sync_copy(rows_v.at[pl.ds(0, rem)],
                            acc_sh.at[pl.ds(base + nfull * Z, rem)])

    @pl.when(s < NS - 1)
    def _():
        _zero_acc(R_MAIN)

    @pl.when(s == NS - 1)
    def _():
        _zero_acc(R_LAST)

    plsc.subcore_barrier()

    cbase = c * SC0_CHUNKS + s * CH_TILE

    def _start(j, buf, sm):
        pltpu.async_copy(x_hbm.at[src_v.at[j]], buf, sm)

    def _finish(j, buf, sm):
        pltpu.make_async_copy(x_hbm.at[src_v.at[j]], buf, sm).wait()
        pltpu.sync_copy(buf, acc_sh.at[dst_v.at[j]], add=True)

    def _run_group(goff, n_ch):
        pltpu.sync_copy(e_hbm.at[0, pl.ds(cbase + goff, n_ch)],
                        src_v.at[pl.ds(0, n_ch)])
        pltpu.sync_copy(e_hbm.at[1, pl.ds(cbase + goff, n_ch)],
                        dst_v.at[pl.ds(0, n_ch)])
        _start(0, rows_v, sem)

        if n_ch > 2:
            @pl.loop(0, n_ch - 2, step=2)
            def _(jb):
                _start(jb + 1, rows2_v, sem2)
                _finish(jb, rows_v, sem)
                _start(jb + 2, rows_v, sem)
                _finish(jb + 1, rows2_v, sem2)

        _start(n_ch - 1, rows2_v, sem2)
        _finish(n_ch - 2, rows_v, sem)
        _finish(n_ch - 1, rows2_v, sem2)

    @pl.when(s < NS - 1)
    def _():
        @pl.loop(0, CH_TILE // G)
        def _(g):
            _run_group(g * G, G)

    @pl.when(s == NS - 1)
    def _():
        @pl.loop(0, 3)
        def _(g):
            _run_group(g * G, G)

        @pl.when(c == 1)
        def _():
            _run_group(3 * G, 4)

    plsc.subcore_barrier()

    @pl.when(s < NS - 1)
    def _():
        pltpu.sync_copy(acc_sh.at[pl.ds(base, R_MAIN)],
                        out_hbm.at[c, pl.ds(base, R_MAIN)])

    @pl.when(s == NS - 1)
    def _():
        pltpu.sync_copy(acc_sh.at[pl.ds(base, R_LAST)],
                        out_hbm.at[c, pl.ds(base, R_LAST)])


@jax.jit
def _sc_aggregate(x, e3):
    mesh = plsc.VectorSubcoreMesh(core_axis_name="c", subcore_axis_name="s")
    return pl.kernel(
        _sc_aggregate_body,
        out_type=jax.ShapeDtypeStruct((NC, N_NODES, F), jnp.float32),
        mesh=mesh,
        scratch_types=[
            pltpu.VMEM_SHARED((N_NODES, F), jnp.float32),
            pltpu.VMEM((G, K), jnp.int32),
            pltpu.VMEM((G, K), jnp.int32),
            pltpu.VMEM((K, F), jnp.float32),
            pltpu.VMEM((K, F), jnp.float32),
            pltpu.SemaphoreType.DMA,
            pltpu.SemaphoreType.DMA,
        ],
    )(x, e3)


def _tc_finish_body(acc_ref, w_ref, b_ref, o_ref):
    a = acc_ref[0] + acc_ref[1]
    y = jnp.dot(a, w_ref[...], preferred_element_type=jnp.float32)
    o_ref[...] = jnp.tanh(y + b_ref[...])


@jax.jit
def _tc_finish(agg, weight, bias):
    blk = 1000
    return pl.pallas_call(
        _tc_finish_body,
        grid=(N_NODES // blk,),
        in_specs=[
            pl.BlockSpec((NC, blk, F), lambda i: (0, i, 0)),
            pl.BlockSpec((F, F), lambda i: (0, 0)),
            pl.BlockSpec((1, F), lambda i: (0, 0)),
        ],
        out_specs=pl.BlockSpec((blk, F), lambda i: (i, 0)),
        out_shape=jax.ShapeDtypeStruct((N_NODES, F), jnp.float32),
    )(agg, weight, bias.reshape(1, F))


def kernel(inputs, edge_index, weight, bias):
    e3 = edge_index.astype(jnp.int32).reshape(2, N_CHUNKS, K)
    agg = _sc_aggregate(inputs, e3)
    return _tc_finish(agg, weight, bias)


# speedup vs baseline: 13.3640x; 1.0184x over previous
import functools

import jax
import jax.numpy as jnp
from jax import lax
from jax.experimental import pallas as pl
from jax.experimental.pallas import tpu as pltpu
from jax.experimental.pallas import tpu_sc as plsc

N_NODES = 10000
N_EDGES = 320000
F = 128

NC = 2
NS = 16
NW = NC * NS

R_MAIN = 632
R_LAST = N_NODES - (NS - 1) * R_MAIN
K = 128
N_CHUNKS = N_EDGES // K
CH_TILE = 80
SC0_CHUNKS = 1248
G = 16
Z = 120


def _sc_aggregate_body(x_hbm, e_hbm, out_hbm,
                       acc_sh, src_v, dst_v, rows_v, rows2_v,
                       sem, sem2):
    c = lax.axis_index("c")
    s = lax.axis_index("s")
    wid = c * NS + s

    zero16 = jnp.zeros((16,), jnp.float32)

    def _zrow(i, carry):
        for k in range(F // 16):
            rows_v[i, pl.ds(k * 16, 16)] = zero16
        return carry

    lax.fori_loop(0, Z, _zrow, 0)

    base = s * R_MAIN

    def _zero_acc(nrows):
        nfull = nrows // Z
        rem = nrows - nfull * Z

        def _zacc(i, carry):
            pltpu.sync_copy(rows_v.at[pl.ds(0, Z)],
                            acc_sh.at[pl.ds(base + i * Z, Z)])
            return carry

        lax.fori_loop(0, nfull, _zacc, 0)
        if rem:
            pltpu.sync_copy(rows_v.at[pl.ds(0, rem)],
                            acc_sh.at[pl.ds(base + nfull * Z, rem)])

    @pl.when(s < NS - 1)
    def _():
        _zero_acc(R_MAIN)

    @pl.when(s == NS - 1)
    def _():
        _zero_acc(R_LAST)

    plsc.subcore_barrier()

    cbase = c * SC0_CHUNKS + s * CH_TILE

    def _start(j, buf, sm):
        pltpu.async_copy(x_hbm.at[src_v.at[j]], buf, sm)

    def _finish(j, buf, sm):
        pltpu.make_async_copy(x_hbm.at[src_v.at[j]], buf, sm).wait()
        pltpu.sync_copy(buf, acc_sh.at[dst_v.at[j]], add=True)

    def _run_group(goff, n_ch):
        pltpu.sync_copy(e_hbm.at[0, pl.ds(cbase + goff, n_ch)],
                        src_v.at[pl.ds(0, n_ch)])
        pltpu.sync_copy(e_hbm.at[1, pl.ds(cbase + goff, n_ch)],
                        dst_v.at[pl.ds(0, n_ch)])
        _start(0, rows_v, sem)

        if n_ch > 2:
            @pl.loop(0, n_ch - 2, step=2)
            def _(jb):
                _start(jb + 1, rows2_v, sem2)
                _finish(jb, rows_v, sem)
                _start(jb + 2, rows_v, sem)
                _finish(jb + 1, rows2_v, sem2)

        _start(n_ch - 1, rows2_v, sem2)
        _finish(n_ch - 2, rows_v, sem)
        _finish(n_ch - 1, rows2_v, sem2)

    @pl.when(s < NS - 1)
    def _():
        @pl.loop(0, CH_TILE // G)
        def _(g):
            _run_group(g * G, G)

    @pl.when(s == NS - 1)
    def _():
        @pl.loop(0, 3)
        def _(g):
            _run_group(g * G, G)

        @pl.when(c == 1)
        def _():
            _run_group(3 * G, 4)

    plsc.subcore_barrier()

    @pl.when(s < NS - 1)
    def _():
        pltpu.sync_copy(acc_sh.at[pl.ds(base, R_MAIN)],
                        out_hbm.at[c, pl.ds(base, R_MAIN)])

    @pl.when(s == NS - 1)
    def _():
        pltpu.sync_copy(acc_sh.at[pl.ds(base, R_LAST)],
                        out_hbm.at[c, pl.ds(base, R_LAST)])


@jax.jit
def _sc_aggregate(x, e3):
    mesh = plsc.VectorSubcoreMesh(core_axis_name="c", subcore_axis_name="s")
    return pl.kernel(
        _sc_aggregate_body,
        out_type=jax.ShapeDtypeStruct((NC, N_NODES, F), jnp.float32),
        mesh=mesh,
        scratch_types=[
            pltpu.VMEM_SHARED((N_NODES, F), jnp.float32),
            pltpu.VMEM((G, K), jnp.int32),
            pltpu.VMEM((G, K), jnp.int32),
            pltpu.VMEM((K, F), jnp.float32),
            pltpu.VMEM((K, F), jnp.float32),
            pltpu.SemaphoreType.DMA,
            pltpu.SemaphoreType.DMA,
        ],
    )(x, e3)


def _tc_finish_body(acc_ref, w_ref, b_ref, o_ref):
    a = acc_ref[0] + acc_ref[1]
    y = jnp.dot(a, w_ref[...], preferred_element_type=jnp.float32)
    o_ref[...] = jnp.tanh(y + b_ref[...])


@jax.jit
def _tc_finish(agg, weight, bias):
    blk = 2000
    return pl.pallas_call(
        _tc_finish_body,
        grid=(N_NODES // blk,),
        in_specs=[
            pl.BlockSpec((NC, blk, F), lambda i: (0, i, 0)),
            pl.BlockSpec((F, F), lambda i: (0, 0)),
            pl.BlockSpec((1, F), lambda i: (0, 0)),
        ],
        out_specs=pl.BlockSpec((blk, F), lambda i: (i, 0)),
        out_shape=jax.ShapeDtypeStruct((N_NODES, F), jnp.float32),
    )(agg, weight, bias.reshape(1, F))


def kernel(inputs, edge_index, weight, bias):
    e3 = edge_index.astype(jnp.int32).reshape(2, N_CHUNKS, K)
    agg = _sc_aggregate(inputs, e3)
    return _tc_finish(agg, weight, bias)


# speedup vs baseline: 14.0465x; 1.0511x over previous
import functools

import jax
import jax.numpy as jnp
from jax import lax
from jax.experimental import pallas as pl
from jax.experimental.pallas import tpu as pltpu
from jax.experimental.pallas import tpu_sc as plsc

N_NODES = 10000
N_EDGES = 320000
F = 128

NC = 2
NS = 16
NW = NC * NS

R_MAIN = 632
R_LAST = N_NODES - (NS - 1) * R_MAIN
K = 128
N_CHUNKS = N_EDGES // K
CH_TILE = 80
SC0_CHUNKS = 1248
G = 16
KS = 64
R = 5
Z = 64


def _sc_aggregate_body(x_hbm, e_hbm, out_hbm, acc_sh, src_v, dst_v,
                       b0, b1, b2, b3, b4, d0, d1, d2, d3, d4,
                       g0, g1, g2, g3, g4, s0, s1, s2, s3, s4):
    bufs = [b0, b1, b2, b3, b4]
    dstks = [d0, d1, d2, d3, d4]
    gsems = [g0, g1, g2, g3, g4]
    ssems = [s0, s1, s2, s3, s4]

    c = lax.axis_index("c")
    s = lax.axis_index("s")

    zero16 = jnp.zeros((16,), jnp.float32)

    def _zrow(i, carry):
        for k in range(F // 16):
            bufs[0][i, pl.ds(k * 16, 16)] = zero16
        return carry

    lax.fori_loop(0, Z, _zrow, 0)

    base = s * R_MAIN

    def _zero_acc(nrows):
        nfull = nrows // Z
        rem = nrows - nfull * Z

        def _zacc(i, carry):
            pltpu.sync_copy(bufs[0].at[pl.ds(0, Z)],
                            acc_sh.at[pl.ds(base + i * Z, Z)])
            return carry

        lax.fori_loop(0, nfull, _zacc, 0)
        if rem:
            pltpu.sync_copy(bufs[0].at[pl.ds(0, rem)],
                            acc_sh.at[pl.ds(base + nfull * Z, rem)])

    @pl.when(s < NS - 1)
    def _():
        _zero_acc(R_MAIN)

    @pl.when(s == NS - 1)
    def _():
        _zero_acc(R_LAST)

    plsc.subcore_barrier()

    cbase = c * SC0_CHUNKS + s * CH_TILE

    def _gref(u):
        j, h = divmod(u, 2)
        return x_hbm.at[src_v.at[j, pl.ds(h * KS, KS)]]

    def _run_group(goff, n_ch):
        pltpu.sync_copy(e_hbm.at[0, pl.ds(cbase + goff, n_ch)],
                        src_v.at[pl.ds(0, n_ch)])
        pltpu.sync_copy(e_hbm.at[1, pl.ds(cbase + goff, n_ch)],
                        dst_v.at[pl.ds(0, n_ch)])
        nsub = n_ch * 2
        for t in range(nsub + 2):
            if t < nsub:
                u, b = t, t % R
                if u >= R:
                    pltpu.make_async_copy(
                        bufs[b], acc_sh.at[dstks[b]], ssems[b]).wait()
                pltpu.async_copy(_gref(u), bufs[b], gsems[b])
            if t >= 2:
                u, b = t - 2, (t - 2) % R
                pltpu.make_async_copy(_gref(u), bufs[b], gsems[b]).wait()
                j, h = divmod(u, 2)
                for i in range(KS // 16):
                    dstks[b][pl.ds(i * 16, 16)] = (
                        dst_v[j, pl.ds(h * KS + i * 16, 16)])
                pltpu.async_copy(bufs[b], acc_sh.at[dstks[b]], ssems[b],
                                 add=True)
        for u in range(max(0, nsub - R), nsub):
            b = u % R
            pltpu.make_async_copy(bufs[b], acc_sh.at[dstks[b]],
                                  ssems[b]).wait()

    @pl.when(s < NS - 1)
    def _():
        @pl.loop(0, CH_TILE // G)
        def _(g):
            _run_group(g * G, G)

    @pl.when(s == NS - 1)
    def _():
        @pl.loop(0, 3)
        def _(g):
            _run_group(g * G, G)

        @pl.when(c == 1)
        def _():
            _run_group(3 * G, 4)

    plsc.subcore_barrier()

    @pl.when(s < NS - 1)
    def _():
        pltpu.sync_copy(acc_sh.at[pl.ds(base, R_MAIN)],
                        out_hbm.at[c, pl.ds(base, R_MAIN)])

    @pl.when(s == NS - 1)
    def _():
        pltpu.sync_copy(acc_sh.at[pl.ds(base, R_LAST)],
                        out_hbm.at[c, pl.ds(base, R_LAST)])


@jax.jit
def _sc_aggregate(x, e3):
    mesh = plsc.VectorSubcoreMesh(core_axis_name="c", subcore_axis_name="s")
    return pl.kernel(
        _sc_aggregate_body,
        out_type=jax.ShapeDtypeStruct((NC, N_NODES, F), jnp.float32),
        mesh=mesh,
        scratch_types=[
            pltpu.VMEM_SHARED((N_NODES, F), jnp.float32),
            pltpu.VMEM((G, K), jnp.int32),
            pltpu.VMEM((G, K), jnp.int32),
        ] + [pltpu.VMEM((KS, F), jnp.float32)] * R
          + [pltpu.VMEM((KS,), jnp.int32)] * R
          + [pltpu.SemaphoreType.DMA] * (2 * R),
    )(x, e3)


def _tc_finish_body(acc_ref, w_ref, b_ref, o_ref):
    a = acc_ref[0] + acc_ref[1]
    y = jnp.dot(a, w_ref[...], preferred_element_type=jnp.float32)
    o_ref[...] = jnp.tanh(y + b_ref[...])


@jax.jit
def _tc_finish(agg, weight, bias):
    blk = 2000
    return pl.pallas_call(
        _tc_finish_body,
        grid=(N_NODES // blk,),
        in_specs=[
            pl.BlockSpec((NC, blk, F), lambda i: (0, i, 0)),
            pl.BlockSpec((F, F), lambda i: (0, 0)),
            pl.BlockSpec((1, F), lambda i: (0, 0)),
        ],
        out_specs=pl.BlockSpec((blk, F), lambda i: (i, 0)),
        out_shape=jax.ShapeDtypeStruct((N_NODES, F), jnp.float32),
    )(agg, weight, bias.reshape(1, F))


def kernel(inputs, edge_index, weight, bias):
    e3 = edge_index.astype(jnp.int32).reshape(2, N_CHUNKS, K)
    agg = _sc_aggregate(inputs, e3)
    return _tc_finish(agg, weight, bias)
